# baseline (speedup 1.0000x reference)
import functools

import jax
import jax.numpy as jnp
from jax import lax
from jax.experimental import pallas as pl
from jax.experimental.pallas import tpu as pltpu
from jax.experimental.pallas import tpu_sc as plsc

_N = 4096
_LO, _HI = 1024, 3072
_NC, _NS = 2, 16
_NW = _NC * _NS
_RPW = _N // _NW
_CH = 8

_mesh = plsc.VectorSubcoreMesh(core_axis_name="c", subcore_axis_name="s")


@functools.partial(
    pl.kernel,
    out_type=jax.ShapeDtypeStruct((1, _N, _N), jnp.float32),
    mesh=_mesh,
    scratch_types=[
        pltpu.VMEM((_CH, _N), jnp.float32),
        pltpu.VMEM((16,), jnp.float32),
    ],
)
def _sc_body(b_hbm, bt_hbm, o_hbm, buf, btv):
    wid = lax.axis_index("s") * _NC + lax.axis_index("c")
    base = wid * _RPW
    in_band = (base >= _LO) & (base < _HI)

    @pl.when(jnp.logical_not(in_band))
    def _outside():
        for k in range(_RPW // _CH):
            r = base + k * _CH
            pltpu.sync_copy(b_hbm.at[0, pl.ds(r, _CH), :], buf)
            pltpu.sync_copy(buf, o_hbm.at[0, pl.ds(r, _CH), :])

    @pl.when(in_band)
    def _band():
        pltpu.sync_copy(bt_hbm, btv)
        splat = btv[...]

        def _fill(c, carry):
            for rr in range(_CH):
                buf[rr, pl.ds(_LO + c * 16, 16)] = splat
            return carry

        lax.fori_loop(0, (_HI - _LO) // 16, _fill, 0)
        for k in range(_RPW // _CH):
            r = base + k * _CH
            pltpu.sync_copy(b_hbm.at[0, pl.ds(r, _CH), pl.ds(0, _LO)],
                            buf.at[:, pl.ds(0, _LO)])
            pltpu.sync_copy(b_hbm.at[0, pl.ds(r, _CH), pl.ds(_HI, _N - _HI)],
                            buf.at[:, pl.ds(_HI, _N - _HI)])
            pltpu.sync_copy(buf, o_hbm.at[0, pl.ds(r, _CH), :])


def kernel(B, Bt):
    bt16 = jnp.broadcast_to(jnp.reshape(Bt, (1,)), (16,))
    return _sc_body(B, bt16)


# speedup vs baseline: 1.2774x; 1.2774x over previous
import functools

import jax
import jax.numpy as jnp
from jax import lax
from jax.experimental import pallas as pl
from jax.experimental.pallas import tpu as pltpu
from jax.experimental.pallas import tpu_sc as plsc

_N = 4096
_LO, _HI = 1024, 3072
_NC, _NS = 2, 16
_NW = _NC * _NS
_RPW = _N // _NW
_CH = 8
_NB = 3
_NCHUNK = _RPW // _CH

_mesh = plsc.VectorSubcoreMesh(core_axis_name="c", subcore_axis_name="s")


@functools.partial(
    pl.kernel,
    out_type=jax.ShapeDtypeStruct((1, _N, _N), jnp.float32),
    mesh=_mesh,
    scratch_types=[
        [pltpu.VMEM((_CH, _N), jnp.float32) for _ in range(_NB)],
        pltpu.VMEM((16,), jnp.float32),
        pltpu.SemaphoreType.DMA((_NB,)),
        pltpu.SemaphoreType.DMA((_NB,)),
    ],
)
def _sc_body(b_hbm, bt_hbm, o_hbm, bufs, btv, sin, sout):
    wid = lax.axis_index("s") * _NC + lax.axis_index("c")
    base = wid * _RPW
    in_band = (base >= _LO) & (base < _HI)

    def _run(read_of):
        def write_of(k):
            b = k % _NB
            return [pltpu.make_async_copy(
                bufs[b], o_hbm.at[0, pl.ds(base + k * _CH, _CH), :],
                sout.at[b])]

        for k in range(min(_NB, _NCHUNK)):
            for c in read_of(k):
                c.start()
        for k in range(_NCHUNK):
            for c in read_of(k):
                c.wait()
            for c in write_of(k):
                c.start()
            if k + _NB < _NCHUNK:
                for c in write_of(k):
                    c.wait()
                for c in read_of(k + _NB):
                    c.start()
        for k in range(max(_NCHUNK - _NB, 0), _NCHUNK):
            for c in write_of(k):
                c.wait()

    @pl.when(jnp.logical_not(in_band))
    def _outside():
        def read_of(k):
            b = k % _NB
            return [pltpu.make_async_copy(
                b_hbm.at[0, pl.ds(base + k * _CH, _CH), :], bufs[b],
                sin.at[b])]

        _run(read_of)

    @pl.when(in_band)
    def _band():
        pltpu.sync_copy(bt_hbm, btv)
        splat = btv[...]

        def _fill(c, carry):
            for b in range(_NB):
                for rr in range(_CH):
                    bufs[b][rr, pl.ds(_LO + c * 16, 16)] = splat
            return carry

        lax.fori_loop(0, (_HI - _LO) // 16, _fill, 0)

        def read_of(k):
            b = k % _NB
            r = base + k * _CH
            return [
                pltpu.make_async_copy(
                    b_hbm.at[0, pl.ds(r, _CH), pl.ds(0, _LO)],
                    bufs[b].at[:, pl.ds(0, _LO)], sin.at[b]),
                pltpu.make_async_copy(
                    b_hbm.at[0, pl.ds(r, _CH), pl.ds(_HI, _N - _HI)],
                    bufs[b].at[:, pl.ds(_HI, _N - _HI)], sin.at[b]),
            ]

        _run(read_of)


def kernel(B, Bt):
    bt16 = jnp.broadcast_to(jnp.reshape(Bt, (1,)), (16,))
    return _sc_body(B, bt16)
